# baseline (speedup 1.0000x reference)
import functools

import jax
import jax.numpy as jnp
from jax import lax
from jax.experimental import pallas as pl
from jax.experimental.pallas import tpu as pltpu
from jax.experimental.pallas import tpu_sc as plsc

B, S, V, D = 4096, 200, 1000000, 64
NCF, CV, CD = 26, 100000, 16
NCLS = 1000

NUM_CORES, NUM_SUBCORES = 2, 16
NW = NUM_CORES * NUM_SUBCORES
BPW = B // NW
S0, S1 = 96, 104

_sc_mesh = plsc.VectorSubcoreMesh(core_axis_name="c", subcore_axis_name="s")

TBK = 512


def _tr_body(in_ref, out_ref):
    out_ref[:, pl.ds(0, D)] = in_ref[...].T


def _transpose_emb(embT):
    return pl.pallas_call(
        _tr_body,
        grid=(pl.cdiv(V, TBK),),
        in_specs=[pl.BlockSpec((D, TBK), lambda j: (0, j))],
        out_specs=pl.BlockSpec((TBK, 128), lambda j: (j, 0)),
        out_shape=jax.ShapeDtypeStruct((V, 128), jnp.float32),
    )(embT)


@functools.partial(
    pl.kernel,
    out_type=[
        jax.ShapeDtypeStruct((B, D), jnp.float32),
        jax.ShapeDtypeStruct((NCF, B, CD), jnp.float32),
    ],
    mesh=_sc_mesh,
    compiler_params=pltpu.CompilerParams(use_tc_tiling_on_sc=False),
    scratch_types=[
        pltpu.VMEM((BPW, S), jnp.int32),
        pltpu.VMEM((S, 128), jnp.float32),
        pltpu.VMEM((S, 128), jnp.float32),
        pltpu.VMEM((BPW, D), jnp.float32),
        pltpu.VMEM((NCF, BPW), jnp.int32),
        pltpu.VMEM((NCF // 2, BPW, CD), jnp.float32),
        pltpu.SemaphoreType.DMA,
        pltpu.SemaphoreType.DMA,
        pltpu.SemaphoreType.DMA,
    ],
)
def _sc_embed(tidx_hbm, emb_hbm, cvars_hbm, cat_hbm, xt_hbm, xc3_hbm,
              tidx_v, buf_a, buf_b, xt_v, cidx_v, xc_v,
              sem_a, sem_b, sem_c):
    wid = lax.axis_index("s") * NUM_CORES + lax.axis_index("c")
    tbase = wid * BPW

    pltpu.sync_copy(tidx_hbm.at[pl.ds(tbase, BPW)], tidx_v)

    pltpu.sync_copy(cvars_hbm.at[pl.ds(0, NCF), pl.ds(tbase, BPW)], cidx_v)

    NCH = NCF // 2
    cat_handles = []
    for c in range(NCH):
        cat_handles.append(pltpu.async_copy(
            cat_hbm.at[c].at[cidx_v.at[c]], xc_v.at[c], sem_c))

    def issue(buf, sem, r):
        pltpu.async_copy(emb_hbm.at[tidx_v.at[r, pl.ds(0, S0)]],
                         buf.at[pl.ds(0, S0)], sem)
        pltpu.async_copy(emb_hbm.at[tidx_v.at[r, pl.ds(S0, S1)]],
                         buf.at[pl.ds(S0, S1)], sem)

    def drain(buf, sem):
        pltpu.make_async_copy(emb_hbm.at[pl.ds(0, S0)],
                              buf.at[pl.ds(0, S0)], sem).wait()
        pltpu.make_async_copy(emb_hbm.at[pl.ds(0, S1)],
                              buf.at[pl.ds(S0, S1)], sem).wait()

    RCHUNK, NCHUNK = 25, S // 25

    def reduce_into(buf, r):
        def chunk(c, tots):
            base = c * RCHUNK
            t = list(tots)
            for g in range(RCHUNK):
                for j in range(4):
                    t[j] = t[j] + buf[base + g, pl.ds(16 * j, 16)]
            return tuple(t)

        z = jnp.zeros((16,), jnp.float32)
        tots = lax.fori_loop(0, NCHUNK, chunk, (z, z, z, z))
        for j in range(4):
            xt_v[r, pl.ds(16 * j, 16)] = tots[j]

    issue(buf_a, sem_a, 0)
    issue(buf_b, sem_b, 1)

    def row_body(k, carry):
        r = 2 * k
        drain(buf_a, sem_a)
        reduce_into(buf_a, r)
        issue(buf_a, sem_a, r + 2)
        drain(buf_b, sem_b)
        reduce_into(buf_b, r + 1)
        issue(buf_b, sem_b, r + 3)
        return carry

    lax.fori_loop(0, BPW // 2 - 1, row_body, 0)
    drain(buf_a, sem_a)
    reduce_into(buf_a, BPW - 2)
    drain(buf_b, sem_b)
    reduce_into(buf_b, BPW - 1)

    pltpu.sync_copy(xt_v, xt_hbm.at[pl.ds(tbase, BPW)])
    for h in cat_handles:
        h.wait()
    pltpu.sync_copy(xc_v, xc3_hbm.at[pl.ds(0, NCH), pl.ds(tbase, BPW)])
    cat_handles2 = []
    for c in range(NCH, NCF):
        cat_handles2.append(pltpu.async_copy(
            cat_hbm.at[c].at[cidx_v.at[c]], xc_v.at[c - NCH], sem_c))
    for h in cat_handles2:
        h.wait()
    pltpu.sync_copy(xc_v, xc3_hbm.at[pl.ds(NCH, NCH), pl.ds(tbase, BPW)])


def _tc_head(xt_ref, mask_ref, x3_ref, w_ref, b_ref, o_ref):
    denom = jnp.clip(jnp.sum(mask_ref[...], axis=1, keepdims=True), 1.0, None)
    parts = [xt_ref[...] / denom] + [x3_ref[c] for c in range(NCF)]
    x = jnp.concatenate(parts, axis=1)
    acc = lax.dot_general(x, w_ref[...], (((1,), (1,)), ((), ())),
                          preferred_element_type=jnp.float32)
    o_ref[...] = acc + b_ref[...]


BM = 512


def kernel(encoded_text, attention_mask, categorical_vars, emb_table, cat_tables, W, b):
    emb2 = _transpose_emb(emb_table.T)
    xt_sum, xc3 = _sc_embed(encoded_text, emb2, categorical_vars.T, cat_tables)
    b2 = b.reshape(1, NCLS)

    out = pl.pallas_call(
        _tc_head,
        grid=(B // BM,),
        in_specs=[
            pl.BlockSpec((BM, D), lambda i: (i, 0)),
            pl.BlockSpec((BM, S), lambda i: (i, 0)),
            pl.BlockSpec((NCF, BM, CD), lambda i: (0, i, 0)),
            pl.BlockSpec((NCLS, D + NCF * CD), lambda i: (0, 0)),
            pl.BlockSpec((1, NCLS), lambda i: (0, 0)),
        ],
        out_specs=pl.BlockSpec((BM, NCLS), lambda i: (i, 0)),
        out_shape=jax.ShapeDtypeStruct((B, NCLS), jnp.float32),
    )(xt_sum, attention_mask, xc3, W, b2)
    return out


# speedup vs baseline: 1.5943x; 1.5943x over previous
import functools

import jax
import jax.numpy as jnp
from jax import lax
from jax.experimental import pallas as pl
from jax.experimental.pallas import tpu as pltpu
from jax.experimental.pallas import tpu_sc as plsc

B, S, V, D = 4096, 200, 1000000, 64
NCF, CV, CD = 26, 100000, 16
NCLS = 1000

NUM_CORES, NUM_SUBCORES = 2, 16
NW = NUM_CORES * NUM_SUBCORES
BPW = B // NW
S0, S1 = 96, 104

_sc_mesh = plsc.VectorSubcoreMesh(core_axis_name="c", subcore_axis_name="s")

TBK = 8192


def _tr_body(in_ref, out_ref):
    eye = jnp.eye(D, dtype=jnp.float32)
    out_ref[:, pl.ds(0, D)] = lax.dot_general(
        in_ref[...], eye, (((0,), (0,)), ((), ())),
        preferred_element_type=jnp.float32)


def _transpose_emb(embT):
    return pl.pallas_call(
        _tr_body,
        grid=(pl.cdiv(V, TBK),),
        in_specs=[pl.BlockSpec((D, TBK), lambda j: (0, j))],
        out_specs=pl.BlockSpec((TBK, 128), lambda j: (j, 0)),
        out_shape=jax.ShapeDtypeStruct((V, 128), jnp.float32),
    )(embT)


CQ = 5000
CS = CQ // 8


def _ctr_body(in_ref, out_ref):
    eye = jnp.eye(CD, dtype=jnp.float32)
    for h in range(2):
        for q in range(CV // CQ):
            xt = lax.dot_general(
                in_ref[h, :, pl.ds(q * CQ, CQ)], eye, (((0,), (0,)), ((), ())),
                preferred_element_type=jnp.float32)
            packed = jnp.concatenate(
                [xt[k * CS:(k + 1) * CS] for k in range(8)], axis=1)
            out_ref[pl.ds(h * (CV * CD // 128) + q * CS, CS)] = packed


def _transpose_cat(catT):
    return pl.pallas_call(
        _ctr_body,
        grid=(NCF // 2,),
        in_specs=[pl.BlockSpec((2, CD, CV), lambda g: (g, 0, 0))],
        out_specs=pl.BlockSpec((2 * CV * CD // 128, 128), lambda g: (g, 0)),
        out_shape=jax.ShapeDtypeStruct((NCF * CV * CD // 128, 128), jnp.float32),
    )(catT)


@functools.partial(
    pl.kernel,
    out_type=[
        jax.ShapeDtypeStruct((B, D), jnp.float32),
        jax.ShapeDtypeStruct((NCF, B, CD), jnp.float32),
    ],
    mesh=_sc_mesh,
    compiler_params=pltpu.CompilerParams(use_tc_tiling_on_sc=False),
    scratch_types=[
        pltpu.VMEM((BPW, S), jnp.int32),
        pltpu.VMEM((S, D), jnp.float32),
        pltpu.VMEM((S, D), jnp.float32),
        pltpu.VMEM((BPW, D), jnp.float32),
        pltpu.VMEM((NCF, BPW), jnp.int32),
        pltpu.VMEM((NCF, BPW, CD), jnp.float32),
        pltpu.SemaphoreType.DMA,
        pltpu.SemaphoreType.DMA,
        pltpu.SemaphoreType.DMA,
    ],
)
def _sc_embed(tidx_hbm, emb_hbm, cvars_hbm, cat_hbm, xt_hbm, xc3_hbm,
              tidx_v, buf_a, buf_b, xt_v, cidx_v, xc_v,
              sem_a, sem_b, sem_c):
    wid = lax.axis_index("s") * NUM_CORES + lax.axis_index("c")
    tbase = wid * BPW

    pltpu.sync_copy(tidx_hbm.at[pl.ds(tbase, BPW)], tidx_v)

    pltpu.sync_copy(cvars_hbm.at[pl.ds(0, NCF), pl.ds(tbase, BPW)], cidx_v)

    cat_handles = []
    for c in range(NCF):
        cat_handles.append(pltpu.async_copy(
            cat_hbm.at[cidx_v.at[c]], xc_v.at[c], sem_c))

    def issue(buf, sem, r):
        pltpu.async_copy(emb_hbm.at[tidx_v.at[r, pl.ds(0, S0)]],
                         buf.at[pl.ds(0, S0)], sem)
        pltpu.async_copy(emb_hbm.at[tidx_v.at[r, pl.ds(S0, S1)]],
                         buf.at[pl.ds(S0, S1)], sem)

    def drain(buf, sem):
        pltpu.make_async_copy(emb_hbm.at[pl.ds(0, S0)],
                              buf.at[pl.ds(0, S0)], sem).wait()
        pltpu.make_async_copy(emb_hbm.at[pl.ds(0, S1)],
                              buf.at[pl.ds(S0, S1)], sem).wait()

    RCHUNK, NCHUNK = 25, S // 25

    def reduce_into(buf, r):
        def chunk(c, tots):
            base = c * RCHUNK
            t = list(tots)
            for g in range(RCHUNK):
                for j in range(4):
                    t[j] = t[j] + buf[base + g, pl.ds(16 * j, 16)]
            return tuple(t)

        z = jnp.zeros((16,), jnp.float32)
        tots = lax.fori_loop(0, NCHUNK, chunk, (z, z, z, z))
        for j in range(4):
            xt_v[r, pl.ds(16 * j, 16)] = tots[j]

    issue(buf_a, sem_a, 0)
    issue(buf_b, sem_b, 1)

    def row_body(k, carry):
        r = 2 * k
        drain(buf_a, sem_a)
        reduce_into(buf_a, r)
        issue(buf_a, sem_a, r + 2)
        drain(buf_b, sem_b)
        reduce_into(buf_b, r + 1)
        issue(buf_b, sem_b, r + 3)
        return carry

    lax.fori_loop(0, BPW // 2 - 1, row_body, 0)
    drain(buf_a, sem_a)
    reduce_into(buf_a, BPW - 2)
    drain(buf_b, sem_b)
    reduce_into(buf_b, BPW - 1)

    pltpu.sync_copy(xt_v, xt_hbm.at[pl.ds(tbase, BPW)])
    for h in cat_handles:
        h.wait()
    pltpu.sync_copy(xc_v, xc3_hbm.at[pl.ds(0, NCF), pl.ds(tbase, BPW)])


def _tc_head(xt_ref, mask_ref, x3_ref, w_ref, b_ref, o_ref):
    denom = jnp.clip(jnp.sum(mask_ref[...], axis=1, keepdims=True), 1.0, None)
    parts = [xt_ref[...] / denom] + [x3_ref[c] for c in range(NCF)]
    x = jnp.concatenate(parts, axis=1)
    acc = lax.dot_general(x, w_ref[...], (((1,), (1,)), ((), ())),
                          preferred_element_type=jnp.float32)
    o_ref[...] = acc + b_ref[...]


BM = 512


def kernel(encoded_text, attention_mask, categorical_vars, emb_table, cat_tables, W, b):
    emb2 = _transpose_emb(emb_table.T).reshape(2 * V, D)
    cat2 = _transpose_cat(cat_tables.transpose(0, 2, 1)).reshape(NCF * CV, CD)
    v = categorical_vars.T
    cidx = ((jnp.arange(NCF, dtype=jnp.int32) * CV)[:, None]
            + (v // CQ) * CQ + (v % CS) * 8 + (v % CQ) // CS)
    xt_sum, xc3 = _sc_embed(encoded_text * 2, emb2, cidx, cat2)
    b2 = b.reshape(1, NCLS)

    out = pl.pallas_call(
        _tc_head,
        grid=(B // BM,),
        in_specs=[
            pl.BlockSpec((BM, D), lambda i: (i, 0)),
            pl.BlockSpec((BM, S), lambda i: (i, 0)),
            pl.BlockSpec((NCF, BM, CD), lambda i: (0, i, 0)),
            pl.BlockSpec((NCLS, D + NCF * CD), lambda i: (0, 0)),
            pl.BlockSpec((1, NCLS), lambda i: (0, 0)),
        ],
        out_specs=pl.BlockSpec((BM, NCLS), lambda i: (i, 0)),
        out_shape=jax.ShapeDtypeStruct((B, NCLS), jnp.float32),
    )(xt_sum, attention_mask, xc3, W, b2)
    return out


# speedup vs baseline: 3.9224x; 2.4603x over previous
import functools

import jax
import jax.numpy as jnp
from jax import lax
from jax.experimental import pallas as pl
from jax.experimental.pallas import tpu as pltpu
from jax.experimental.pallas import tpu_sc as plsc

B, S, V, D = 4096, 200, 1000000, 64
NCF, CV, CD = 26, 100000, 16
NCLS = 1000

NUM_CORES, NUM_SUBCORES = 2, 16
NW = NUM_CORES * NUM_SUBCORES
BPW = B // NW
S0, S1 = 96, 104

_sc_mesh = plsc.VectorSubcoreMesh(core_axis_name="c", subcore_axis_name="s")

TBK = 8192


def _tr_body(in_ref, out_ref):
    eye = jnp.eye(D, dtype=jnp.float32)
    out_ref[:, pl.ds(0, D)] = lax.dot_general(
        in_ref[...], eye, (((0,), (0,)), ((), ())),
        preferred_element_type=jnp.float32)


def _transpose_emb(embT):
    return pl.pallas_call(
        _tr_body,
        grid=(pl.cdiv(V, TBK),),
        in_specs=[pl.BlockSpec((D, TBK), lambda j: (0, j))],
        out_specs=pl.BlockSpec((TBK, 128), lambda j: (j, 0)),
        out_shape=jax.ShapeDtypeStruct((V, 128), jnp.float32),
    )(embT)


CQ = 4096
CS = CQ // 8
NQ = CV // CQ
CT = CV - NQ * CQ
CST = CT // 8
CPAD = NQ * CS + CST + 4


def _pack_dot(x):
    s = x.shape[1] // 8
    xs = jnp.concatenate([x[:, k * s:(k + 1) * s] for k in range(8)], axis=0)
    return lax.dot_general(xs, jnp.eye(128, dtype=jnp.float32),
                           (((0,), (0,)), ((), ())),
                           preferred_element_type=jnp.float32)


def _ctr_body(in_ref, out_ref):
    for h in range(2):
        for q in range(NQ):
            out_ref[h, pl.ds(q * CS, CS)] = _pack_dot(
                in_ref[h, :, pl.ds(q * CQ, CQ)])
        out_ref[h, pl.ds(NQ * CS, CST)] = _pack_dot(
            in_ref[h, :, pl.ds(NQ * CQ, CT)])


def _transpose_cat(catT):
    return pl.pallas_call(
        _ctr_body,
        grid=(NCF // 2,),
        in_specs=[pl.BlockSpec((2, CD, CV), lambda g: (g, 0, 0))],
        out_specs=pl.BlockSpec((2, CPAD, 128), lambda g: (g, 0, 0)),
        out_shape=jax.ShapeDtypeStruct((NCF, CPAD, 128), jnp.float32),
    )(catT)


@functools.partial(
    pl.kernel,
    out_type=[
        jax.ShapeDtypeStruct((B, D), jnp.float32),
        jax.ShapeDtypeStruct((NCF, B, CD), jnp.float32),
    ],
    mesh=_sc_mesh,
    compiler_params=pltpu.CompilerParams(use_tc_tiling_on_sc=False),
    scratch_types=[
        pltpu.VMEM((BPW, S), jnp.int32),
        pltpu.VMEM((S, D), jnp.float32),
        pltpu.VMEM((S, D), jnp.float32),
        pltpu.VMEM((BPW, D), jnp.float32),
        pltpu.VMEM((NCF, BPW), jnp.int32),
        pltpu.VMEM((NCF, BPW, CD), jnp.float32),
        pltpu.SemaphoreType.DMA,
        pltpu.SemaphoreType.DMA,
        pltpu.SemaphoreType.DMA,
    ],
)
def _sc_embed(tidx_hbm, emb_hbm, cvars_hbm, cat_hbm, xt_hbm, xc3_hbm,
              tidx_v, buf_a, buf_b, xt_v, cidx_v, xc_v,
              sem_a, sem_b, sem_c):
    wid = lax.axis_index("s") * NUM_CORES + lax.axis_index("c")
    tbase = wid * BPW

    pltpu.sync_copy(tidx_hbm.at[pl.ds(tbase, BPW)], tidx_v)

    pltpu.sync_copy(cvars_hbm.at[pl.ds(0, NCF), pl.ds(tbase, BPW)], cidx_v)

    cat_handles = []
    for c in range(NCF):
        cat_handles.append(pltpu.async_copy(
            cat_hbm.at[cidx_v.at[c]], xc_v.at[c], sem_c))

    def issue(buf, sem, r):
        pltpu.async_copy(emb_hbm.at[tidx_v.at[r, pl.ds(0, S0)]],
                         buf.at[pl.ds(0, S0)], sem)
        pltpu.async_copy(emb_hbm.at[tidx_v.at[r, pl.ds(S0, S1)]],
                         buf.at[pl.ds(S0, S1)], sem)

    def drain(buf, sem):
        pltpu.make_async_copy(emb_hbm.at[pl.ds(0, S0)],
                              buf.at[pl.ds(0, S0)], sem).wait()
        pltpu.make_async_copy(emb_hbm.at[pl.ds(0, S1)],
                              buf.at[pl.ds(S0, S1)], sem).wait()

    RCHUNK, NCHUNK = 25, S // 25

    def reduce_into(buf, r):
        def chunk(c, tots):
            base = c * RCHUNK
            t = list(tots)
            for g in range(RCHUNK):
                for j in range(4):
                    t[j] = t[j] + buf[base + g, pl.ds(16 * j, 16)]
            return tuple(t)

        z = jnp.zeros((16,), jnp.float32)
        tots = lax.fori_loop(0, NCHUNK, chunk, (z, z, z, z))
        for j in range(4):
            xt_v[r, pl.ds(16 * j, 16)] = tots[j]

    issue(buf_a, sem_a, 0)
    issue(buf_b, sem_b, 1)

    def row_body(k, carry):
        r = 2 * k
        drain(buf_a, sem_a)
        reduce_into(buf_a, r)
        issue(buf_a, sem_a, r + 2)
        drain(buf_b, sem_b)
        reduce_into(buf_b, r + 1)
        issue(buf_b, sem_b, r + 3)
        return carry

    lax.fori_loop(0, BPW // 2 - 1, row_body, 0)
    drain(buf_a, sem_a)
    reduce_into(buf_a, BPW - 2)
    drain(buf_b, sem_b)
    reduce_into(buf_b, BPW - 1)

    pltpu.sync_copy(xt_v, xt_hbm.at[pl.ds(tbase, BPW)])
    for h in cat_handles:
        h.wait()
    pltpu.sync_copy(xc_v, xc3_hbm.at[pl.ds(0, NCF), pl.ds(tbase, BPW)])


def _tc_head(xt_ref, mask_ref, x3_ref, w_ref, b_ref, o_ref):
    denom = jnp.clip(jnp.sum(mask_ref[...], axis=1, keepdims=True), 1.0, None)
    parts = [xt_ref[...] / denom] + [x3_ref[c] for c in range(NCF)]
    x = jnp.concatenate(parts, axis=1)
    acc = lax.dot_general(x, w_ref[...], (((1,), (1,)), ((), ())),
                          preferred_element_type=jnp.float32)
    o_ref[...] = acc + b_ref[...]


BM = 512


def kernel(encoded_text, attention_mask, categorical_vars, emb_table, cat_tables, W, b):
    emb2 = _transpose_emb(emb_table.T).reshape(2 * V, D)
    cat2 = _transpose_cat(cat_tables.transpose(0, 2, 1)).reshape(NCF * CPAD * 8, CD)
    v = categorical_vars.T
    q = v // CQ
    u = v - q * CQ
    tail = q >= NQ
    row = jnp.where(tail, NQ * CS + u % CST, q * CS + (u & (CS - 1)))
    k = jnp.where(tail, u // CST, u // CS)
    cidx = (jnp.arange(NCF, dtype=jnp.int32) * (CPAD * 8))[:, None] + row * 8 + k
    xt_sum, xc3 = _sc_embed(encoded_text * 2, emb2, cidx, cat2)
    b2 = b.reshape(1, NCLS)

    out = pl.pallas_call(
        _tc_head,
        grid=(B // BM,),
        in_specs=[
            pl.BlockSpec((BM, D), lambda i: (i, 0)),
            pl.BlockSpec((BM, S), lambda i: (i, 0)),
            pl.BlockSpec((NCF, BM, CD), lambda i: (0, i, 0)),
            pl.BlockSpec((NCLS, D + NCF * CD), lambda i: (0, 0)),
            pl.BlockSpec((1, NCLS), lambda i: (0, 0)),
        ],
        out_specs=pl.BlockSpec((BM, NCLS), lambda i: (i, 0)),
        out_shape=jax.ShapeDtypeStruct((B, NCLS), jnp.float32),
    )(xt_sum, attention_mask, xc3, W, b2)
    return out


# speedup vs baseline: 4.4937x; 1.1457x over previous
import functools

import jax
import jax.numpy as jnp
from jax import lax
from jax.experimental import pallas as pl
from jax.experimental.pallas import tpu as pltpu
from jax.experimental.pallas import tpu_sc as plsc

B, S, V, D = 4096, 200, 1000000, 64
NCF, CV, CD = 26, 100000, 16
NCLS = 1000

NUM_CORES, NUM_SUBCORES = 2, 16
NW = NUM_CORES * NUM_SUBCORES
BPW = B // NW
S0, S1 = 96, 104

_sc_mesh = plsc.VectorSubcoreMesh(core_axis_name="c", subcore_axis_name="s")

TBK = 8192
HBK = TBK // 2
NTB = (V + TBK - 1) // TBK


def _tr_body(in_ref, out_ref):
    x = in_ref[...]
    xs = jnp.concatenate([x[:, :HBK], x[:, HBK:]], axis=0)
    out_ref[...] = lax.dot_general(
        xs, jnp.eye(128, dtype=jnp.float32), (((0,), (0,)), ((), ())),
        preferred_element_type=jnp.float32)


def _transpose_emb(embT):
    return pl.pallas_call(
        _tr_body,
        grid=(NTB,),
        in_specs=[pl.BlockSpec((D, TBK), lambda j: (0, j))],
        out_specs=pl.BlockSpec((HBK, 128), lambda j: (j, 0)),
        out_shape=jax.ShapeDtypeStruct((NTB * HBK, 128), jnp.float32),
    )(embT)


CQ = 4096
CS = CQ // 8
NQ = CV // CQ
CT = CV - NQ * CQ
CST = CT // 8
CPAD = NQ * CS + CST + 4


def _pack_dot(x):
    s = x.shape[1] // 8
    xs = jnp.concatenate([x[:, k * s:(k + 1) * s] for k in range(8)], axis=0)
    return lax.dot_general(xs, jnp.eye(128, dtype=jnp.float32),
                           (((0,), (0,)), ((), ())),
                           preferred_element_type=jnp.float32)


def _ctr_body(in_ref, out_ref):
    for h in range(2):
        for q in range(NQ):
            out_ref[h, pl.ds(q * CS, CS)] = _pack_dot(
                in_ref[h, :, pl.ds(q * CQ, CQ)])
        out_ref[h, pl.ds(NQ * CS, CST)] = _pack_dot(
            in_ref[h, :, pl.ds(NQ * CQ, CT)])


def _transpose_cat(catT):
    return pl.pallas_call(
        _ctr_body,
        grid=(NCF // 2,),
        in_specs=[pl.BlockSpec((2, CD, CV), lambda g: (g, 0, 0))],
        out_specs=pl.BlockSpec((2, CPAD, 128), lambda g: (g, 0, 0)),
        out_shape=jax.ShapeDtypeStruct((NCF, CPAD, 128), jnp.float32),
    )(catT)


@functools.partial(
    pl.kernel,
    out_type=[
        jax.ShapeDtypeStruct((B, D), jnp.float32),
        jax.ShapeDtypeStruct((NCF, B, CD), jnp.float32),
    ],
    mesh=_sc_mesh,
    compiler_params=pltpu.CompilerParams(use_tc_tiling_on_sc=False),
    scratch_types=[
        pltpu.VMEM((BPW, S), jnp.int32),
        pltpu.VMEM((S, D), jnp.float32),
        pltpu.VMEM((S, D), jnp.float32),
        pltpu.VMEM((BPW, D), jnp.float32),
        pltpu.VMEM((NCF, BPW), jnp.int32),
        pltpu.VMEM((NCF, BPW, CD), jnp.float32),
        pltpu.SemaphoreType.DMA,
        pltpu.SemaphoreType.DMA,
        pltpu.SemaphoreType.DMA,
    ],
)
def _sc_embed(tidx_hbm, emb_hbm, cvars_hbm, cat_hbm, xt_hbm, xc3_hbm,
              tidx_v, buf_a, buf_b, xt_v, cidx_v, xc_v,
              sem_a, sem_b, sem_c):
    wid = lax.axis_index("s") * NUM_CORES + lax.axis_index("c")
    tbase = wid * BPW

    pltpu.sync_copy(tidx_hbm.at[pl.ds(tbase, BPW)], tidx_v)

    pltpu.sync_copy(cvars_hbm.at[pl.ds(0, NCF), pl.ds(tbase, BPW)], cidx_v)

    cat_handles = []
    for c in range(NCF):
        cat_handles.append(pltpu.async_copy(
            cat_hbm.at[cidx_v.at[c]], xc_v.at[c], sem_c))

    def issue(buf, sem, r):
        pltpu.async_copy(emb_hbm.at[tidx_v.at[r, pl.ds(0, S0)]],
                         buf.at[pl.ds(0, S0)], sem)
        pltpu.async_copy(emb_hbm.at[tidx_v.at[r, pl.ds(S0, S1)]],
                         buf.at[pl.ds(S0, S1)], sem)

    def drain(buf, sem):
        pltpu.make_async_copy(emb_hbm.at[pl.ds(0, S0)],
                              buf.at[pl.ds(0, S0)], sem).wait()
        pltpu.make_async_copy(emb_hbm.at[pl.ds(0, S1)],
                              buf.at[pl.ds(S0, S1)], sem).wait()

    RCHUNK, NCHUNK = 25, S // 25

    def reduce_into(buf, r):
        def chunk(c, tots):
            base = c * RCHUNK
            t = list(tots)
            for g in range(RCHUNK):
                for j in range(4):
                    t[j] = t[j] + buf[base + g, pl.ds(16 * j, 16)]
            return tuple(t)

        z = jnp.zeros((16,), jnp.float32)
        tots = lax.fori_loop(0, NCHUNK, chunk, (z, z, z, z))
        for j in range(4):
            xt_v[r, pl.ds(16 * j, 16)] = tots[j]

    issue(buf_a, sem_a, 0)
    issue(buf_b, sem_b, 1)

    def row_body(k, carry):
        r = 2 * k
        drain(buf_a, sem_a)
        reduce_into(buf_a, r)
        issue(buf_a, sem_a, r + 2)
        drain(buf_b, sem_b)
        reduce_into(buf_b, r + 1)
        issue(buf_b, sem_b, r + 3)
        return carry

    lax.fori_loop(0, BPW // 2 - 1, row_body, 0)
    drain(buf_a, sem_a)
    reduce_into(buf_a, BPW - 2)
    drain(buf_b, sem_b)
    reduce_into(buf_b, BPW - 1)

    pltpu.sync_copy(xt_v, xt_hbm.at[pl.ds(tbase, BPW)])
    for h in cat_handles:
        h.wait()
    pltpu.sync_copy(xc_v, xc3_hbm.at[pl.ds(0, NCF), pl.ds(tbase, BPW)])


def _tc_head(xt_ref, mask_ref, x3_ref, w_ref, b_ref, o_ref):
    denom = jnp.clip(jnp.sum(mask_ref[...], axis=1, keepdims=True), 1.0, None)
    parts = [xt_ref[...] / denom] + [x3_ref[c] for c in range(NCF)]
    x = jnp.concatenate(parts, axis=1)
    acc = lax.dot_general(x, w_ref[...], (((1,), (1,)), ((), ())),
                          preferred_element_type=jnp.float32)
    o_ref[...] = acc + b_ref[...]


BM = 512


def kernel(encoded_text, attention_mask, categorical_vars, emb_table, cat_tables, W, b):
    emb2 = _transpose_emb(emb_table.T).reshape(NTB * TBK, D)
    cat2 = _transpose_cat(cat_tables.transpose(0, 2, 1)).reshape(NCF * CPAD * 8, CD)
    v = categorical_vars.T
    q = v // CQ
    u = v - q * CQ
    tail = q >= NQ
    row = jnp.where(tail, NQ * CS + u % CST, q * CS + (u & (CS - 1)))
    k = jnp.where(tail, u // CST, u // CS)
    cidx = (jnp.arange(NCF, dtype=jnp.int32) * (CPAD * 8))[:, None] + row * 8 + k
    t = encoded_text
    etext2 = (t >> 13) * TBK + 2 * (t & (HBK - 1)) + ((t >> 12) & 1)
    xt_sum, xc3 = _sc_embed(etext2, emb2, cidx, cat2)
    b2 = b.reshape(1, NCLS)

    out = pl.pallas_call(
        _tc_head,
        grid=(B // BM,),
        in_specs=[
            pl.BlockSpec((BM, D), lambda i: (i, 0)),
            pl.BlockSpec((BM, S), lambda i: (i, 0)),
            pl.BlockSpec((NCF, BM, CD), lambda i: (0, i, 0)),
            pl.BlockSpec((NCLS, D + NCF * CD), lambda i: (0, 0)),
            pl.BlockSpec((1, NCLS), lambda i: (0, 0)),
        ],
        out_specs=pl.BlockSpec((BM, NCLS), lambda i: (i, 0)),
        out_shape=jax.ShapeDtypeStruct((B, NCLS), jnp.float32),
    )(xt_sum, attention_mask, xc3, W, b2)
    return out


# speedup vs baseline: 4.9057x; 1.0917x over previous
import functools

import jax
import jax.numpy as jnp
from jax import lax
from jax.experimental import pallas as pl
from jax.experimental.pallas import tpu as pltpu
from jax.experimental.pallas import tpu_sc as plsc

B, S, V, D = 4096, 200, 1000000, 64
NCF, CV, CD = 26, 100000, 16
NCLS = 1000

NUM_CORES, NUM_SUBCORES = 2, 16
NW = NUM_CORES * NUM_SUBCORES
BPW = B // NW
S0, S1 = 96, 104

_sc_mesh = plsc.VectorSubcoreMesh(core_axis_name="c", subcore_axis_name="s")

TBK = 8192
HBK = TBK // 2
NTB = (V + TBK - 1) // TBK


def _tr_body(in_ref, out_ref):
    x = in_ref[...]
    xs = jnp.concatenate([x[:, :HBK], x[:, HBK:]], axis=0)
    out_ref[...] = lax.dot_general(
        xs, jnp.eye(128, dtype=jnp.float32), (((0,), (0,)), ((), ())),
        preferred_element_type=jnp.float32)


def _transpose_emb(embT):
    return pl.pallas_call(
        _tr_body,
        grid=(NTB,),
        in_specs=[pl.BlockSpec((D, TBK), lambda j: (0, j))],
        out_specs=pl.BlockSpec((HBK, 128), lambda j: (j, 0)),
        out_shape=jax.ShapeDtypeStruct((NTB * HBK, 128), jnp.float32),
    )(embT)


CQ = 4096
CS = CQ // 8
NQ = CV // CQ
CT = CV - NQ * CQ
CST = CT // 8
CPAD = NQ * CS + CST + 4


def _pack_dot(x):
    s = x.shape[1] // 8
    xs = jnp.concatenate([x[:, k * s:(k + 1) * s] for k in range(8)], axis=0)
    return lax.dot_general(xs, jnp.eye(128, dtype=jnp.float32),
                           (((0,), (0,)), ((), ())),
                           preferred_element_type=jnp.float32)


def _ctr_body(in_ref, out_ref):
    for h in range(2):
        for q in range(NQ):
            out_ref[h, pl.ds(q * CS, CS)] = _pack_dot(
                in_ref[h, :, pl.ds(q * CQ, CQ)])
        out_ref[h, pl.ds(NQ * CS, CST)] = _pack_dot(
            in_ref[h, :, pl.ds(NQ * CQ, CT)])


def _transpose_cat(catT):
    return pl.pallas_call(
        _ctr_body,
        grid=(NCF // 2,),
        in_specs=[pl.BlockSpec((2, CD, CV), lambda g: (g, 0, 0))],
        out_specs=pl.BlockSpec((2, CPAD, 128), lambda g: (g, 0, 0)),
        out_shape=jax.ShapeDtypeStruct((NCF, CPAD, 128), jnp.float32),
    )(catT)


@functools.partial(
    pl.kernel,
    out_type=jax.ShapeDtypeStruct((NCF, B, CD), jnp.float32),
    mesh=_sc_mesh,
    compiler_params=pltpu.CompilerParams(use_tc_tiling_on_sc=False),
    scratch_types=[
        pltpu.VMEM((NCF, BPW), jnp.int32),
        pltpu.VMEM((NCF, BPW, CD), jnp.float32),
        pltpu.SemaphoreType.DMA,
    ],
)
def _sc_cat(cvars_hbm, cat_hbm, xc3_hbm, cidx_v, xc_v, sem_c):
    wid = lax.axis_index("s") * NUM_CORES + lax.axis_index("c")
    tbase = wid * BPW
    pltpu.sync_copy(cvars_hbm.at[pl.ds(0, NCF), pl.ds(tbase, BPW)], cidx_v)
    cat_handles = []
    for c in range(NCF):
        cat_handles.append(pltpu.async_copy(
            cat_hbm.at[cidx_v.at[c]], xc_v.at[c], sem_c))
    for h in cat_handles:
        h.wait()
    pltpu.sync_copy(xc_v, xc3_hbm.at[pl.ds(0, NCF), pl.ds(tbase, BPW)])


@functools.partial(
    pl.kernel,
    out_type=jax.ShapeDtypeStruct((B, D), jnp.float32),
    mesh=_sc_mesh,
    compiler_params=pltpu.CompilerParams(use_tc_tiling_on_sc=False),
    scratch_types=[
        pltpu.VMEM((BPW, S), jnp.int32),
        pltpu.VMEM((S, D), jnp.float32),
        pltpu.VMEM((S, D), jnp.float32),
        pltpu.VMEM((BPW, D), jnp.float32),
        pltpu.SemaphoreType.DMA,
        pltpu.SemaphoreType.DMA,
    ],
)
def _sc_text(tidx_hbm, emb_hbm, xt_hbm,
             tidx_v, buf_a, buf_b, xt_v, sem_a, sem_b):
    wid = lax.axis_index("s") * NUM_CORES + lax.axis_index("c")
    tbase = wid * BPW

    pltpu.sync_copy(tidx_hbm.at[pl.ds(tbase, BPW)], tidx_v)

    def issue(buf, sem, r):
        pltpu.async_copy(emb_hbm.at[tidx_v.at[r, pl.ds(0, S0)]],
                         buf.at[pl.ds(0, S0)], sem)
        pltpu.async_copy(emb_hbm.at[tidx_v.at[r, pl.ds(S0, S1)]],
                         buf.at[pl.ds(S0, S1)], sem)

    def drain(buf, sem):
        pltpu.make_async_copy(emb_hbm.at[pl.ds(0, S0)],
                              buf.at[pl.ds(0, S0)], sem).wait()
        pltpu.make_async_copy(emb_hbm.at[pl.ds(0, S1)],
                              buf.at[pl.ds(S0, S1)], sem).wait()

    RCHUNK, NCHUNK = 25, S // 25

    def reduce_into(buf, r):
        def chunk(c, tots):
            base = c * RCHUNK
            t = list(tots)
            for g in range(RCHUNK):
                for j in range(4):
                    t[j] = t[j] + buf[base + g, pl.ds(16 * j, 16)]
            return tuple(t)

        z = jnp.zeros((16,), jnp.float32)
        tots = lax.fori_loop(0, NCHUNK, chunk, (z, z, z, z))
        for j in range(4):
            xt_v[r, pl.ds(16 * j, 16)] = tots[j]

    issue(buf_a, sem_a, 0)
    issue(buf_b, sem_b, 1)

    def row_body(k, carry):
        r = 2 * k
        drain(buf_a, sem_a)
        reduce_into(buf_a, r)
        issue(buf_a, sem_a, r + 2)
        drain(buf_b, sem_b)
        reduce_into(buf_b, r + 1)
        issue(buf_b, sem_b, r + 3)
        return carry

    lax.fori_loop(0, BPW // 2 - 1, row_body, 0)
    drain(buf_a, sem_a)
    reduce_into(buf_a, BPW - 2)
    drain(buf_b, sem_b)
    reduce_into(buf_b, BPW - 1)

    pltpu.sync_copy(xt_v, xt_hbm.at[pl.ds(tbase, BPW)])


def _tc_head(xt_ref, maskT_ref, x3_ref, wT_ref, b_ref, o_ref):
    denom = jnp.clip(jnp.sum(maskT_ref[...], axis=0), 1.0, None)[:, None]
    parts = [xt_ref[...] / denom] + [x3_ref[c] for c in range(NCF)]
    x = jnp.concatenate(parts, axis=1)
    acc = lax.dot_general(wT_ref[...], x, (((0,), (1,)), ((), ())),
                          preferred_element_type=jnp.float32)
    o_ref[...] = acc + b_ref[...]


BM = 512


def kernel(encoded_text, attention_mask, categorical_vars, emb_table, cat_tables, W, b):
    emb2 = _transpose_emb(emb_table.T).reshape(NTB * TBK, D)
    cat2 = _transpose_cat(cat_tables.transpose(0, 2, 1)).reshape(NCF * CPAD * 8, CD)
    v = categorical_vars.T
    q = v // CQ
    u = v - q * CQ
    tail = q >= NQ
    row = jnp.where(tail, NQ * CS + u % CST, q * CS + (u & (CS - 1)))
    k = jnp.where(tail, u // CST, u // CS)
    cidx = (jnp.arange(NCF, dtype=jnp.int32) * (CPAD * 8))[:, None] + row * 8 + k
    t = encoded_text
    etext2 = (t >> 13) * TBK + 2 * (t & (HBK - 1)) + ((t >> 12) & 1)
    xt_sum = _sc_text(etext2, emb2)
    xc3 = _sc_cat(cidx, cat2)
    b2 = b.reshape(NCLS, 1)

    outT = pl.pallas_call(
        _tc_head,
        grid=(B // BM,),
        in_specs=[
            pl.BlockSpec((BM, D), lambda i: (i, 0)),
            pl.BlockSpec((S, BM), lambda i: (0, i)),
            pl.BlockSpec((NCF, BM, CD), lambda i: (0, i, 0)),
            pl.BlockSpec((D + NCF * CD, NCLS), lambda i: (0, 0)),
            pl.BlockSpec((NCLS, 1), lambda i: (0, 0)),
        ],
        out_specs=pl.BlockSpec((NCLS, BM), lambda i: (0, i)),
        out_shape=jax.ShapeDtypeStruct((NCLS, B), jnp.float32),
    )(xt_sum, attention_mask.T, xc3, W.T, b2)
    return outT.T


# speedup vs baseline: 4.9922x; 1.0176x over previous
import functools

import jax
import jax.numpy as jnp
from jax import lax
from jax.experimental import pallas as pl
from jax.experimental.pallas import tpu as pltpu
from jax.experimental.pallas import tpu_sc as plsc

B, S, V, D = 4096, 200, 1000000, 64
NCF, CV, CD = 26, 100000, 16
NCLS = 1000

NUM_CORES, NUM_SUBCORES = 2, 16
NW = NUM_CORES * NUM_SUBCORES
BPW = B // NW
S0, S1 = 96, 104

_sc_mesh = plsc.VectorSubcoreMesh(core_axis_name="c", subcore_axis_name="s")

TBK = 8192
HBK = TBK // 2
NTB = (V + TBK - 1) // TBK


def _tr_body(in_ref, out_ref):
    x = in_ref[...]
    xs = jnp.concatenate([x[:, :HBK], x[:, HBK:]], axis=0)
    out_ref[...] = lax.dot_general(
        xs, jnp.eye(128, dtype=jnp.float32), (((0,), (0,)), ((), ())),
        preferred_element_type=jnp.float32)


def _transpose_emb(embT):
    return pl.pallas_call(
        _tr_body,
        grid=(NTB,),
        in_specs=[pl.BlockSpec((D, TBK), lambda j: (0, j))],
        out_specs=pl.BlockSpec((HBK, 128), lambda j: (j, 0)),
        out_shape=jax.ShapeDtypeStruct((NTB * HBK, 128), jnp.float32),
    )(embT)


CQ = 4096
CS = CQ // 8
NQ = CV // CQ
CT = CV - NQ * CQ
CST = CT // 8
CPAD = NQ * CS + CST + 4


def _pack_dot(x):
    s = x.shape[1] // 8
    xs = jnp.concatenate([x[:, k * s:(k + 1) * s] for k in range(8)], axis=0)
    return lax.dot_general(xs, jnp.eye(128, dtype=jnp.float32),
                           (((0,), (0,)), ((), ())),
                           preferred_element_type=jnp.float32)


def _ctr_body(in_ref, out_ref):
    for h in range(2):
        for q in range(NQ):
            out_ref[h, pl.ds(q * CS, CS)] = _pack_dot(
                in_ref[h, :, pl.ds(q * CQ, CQ)])
        out_ref[h, pl.ds(NQ * CS, CST)] = _pack_dot(
            in_ref[h, :, pl.ds(NQ * CQ, CT)])


def _transpose_cat(catT):
    return pl.pallas_call(
        _ctr_body,
        grid=(NCF // 2,),
        in_specs=[pl.BlockSpec((2, CD, CV), lambda g: (g, 0, 0))],
        out_specs=pl.BlockSpec((2, CPAD, 128), lambda g: (g, 0, 0)),
        out_shape=jax.ShapeDtypeStruct((NCF, CPAD, 128), jnp.float32),
    )(catT)


@functools.partial(
    pl.kernel,
    out_type=jax.ShapeDtypeStruct((NCF, B, CD), jnp.float32),
    mesh=_sc_mesh,
    compiler_params=pltpu.CompilerParams(use_tc_tiling_on_sc=False),
    scratch_types=[
        pltpu.VMEM((NCF, BPW), jnp.int32),
        pltpu.VMEM((NCF, BPW, CD), jnp.float32),
        pltpu.SemaphoreType.DMA,
    ],
)
def _sc_cat(cvars_hbm, cat_hbm, xc3_hbm, cidx_v, xc_v, sem_c):
    wid = lax.axis_index("s") * NUM_CORES + lax.axis_index("c")
    tbase = wid * BPW
    pltpu.sync_copy(cvars_hbm.at[pl.ds(0, NCF), pl.ds(tbase, BPW)], cidx_v)
    cat_handles = []
    for c in range(NCF):
        cat_handles.append(pltpu.async_copy(
            cat_hbm.at[cidx_v.at[c]], xc_v.at[c], sem_c))
    for h in cat_handles:
        h.wait()
    pltpu.sync_copy(xc_v, xc3_hbm.at[pl.ds(0, NCF), pl.ds(tbase, BPW)])


@functools.partial(
    pl.kernel,
    out_type=jax.ShapeDtypeStruct((B, D), jnp.float32),
    mesh=_sc_mesh,
    compiler_params=pltpu.CompilerParams(use_tc_tiling_on_sc=False),
    scratch_types=[
        pltpu.VMEM((BPW, S), jnp.int32),
        pltpu.VMEM((S, D), jnp.float32),
        pltpu.VMEM((S, D), jnp.float32),
        pltpu.VMEM((S, D), jnp.float32),
        pltpu.VMEM((S, D), jnp.float32),
        pltpu.VMEM((BPW, D), jnp.float32),
        pltpu.SemaphoreType.DMA,
        pltpu.SemaphoreType.DMA,
        pltpu.SemaphoreType.DMA,
        pltpu.SemaphoreType.DMA,
    ],
)
def _sc_text(tidx_hbm, emb_hbm, xt_hbm,
             tidx_v, buf_0, buf_1, buf_2, buf_3, xt_v,
             sem_0, sem_1, sem_2, sem_3):
    wid = lax.axis_index("s") * NUM_CORES + lax.axis_index("c")
    tbase = wid * BPW

    pltpu.sync_copy(tidx_hbm.at[pl.ds(tbase, BPW)], tidx_v)

    def issue(buf, sem, r):
        pltpu.async_copy(emb_hbm.at[tidx_v.at[r, pl.ds(0, S0)]],
                         buf.at[pl.ds(0, S0)], sem)
        pltpu.async_copy(emb_hbm.at[tidx_v.at[r, pl.ds(S0, S1)]],
                         buf.at[pl.ds(S0, S1)], sem)

    def drain(buf, sem):
        pltpu.make_async_copy(emb_hbm.at[pl.ds(0, S0)],
                              buf.at[pl.ds(0, S0)], sem).wait()
        pltpu.make_async_copy(emb_hbm.at[pl.ds(0, S1)],
                              buf.at[pl.ds(S0, S1)], sem).wait()

    RCHUNK, NCHUNK = 25, S // 25

    def reduce_into(buf, r):
        def chunk(c, tots):
            base = c * RCHUNK
            t = list(tots)
            for g in range(RCHUNK):
                for j in range(4):
                    t[j] = t[j] + buf[base + g, pl.ds(16 * j, 16)]
            return tuple(t)

        z = jnp.zeros((16,), jnp.float32)
        tots = lax.fori_loop(0, NCHUNK, chunk, (z, z, z, z))
        for j in range(4):
            xt_v[r, pl.ds(16 * j, 16)] = tots[j]

    bufs = (buf_0, buf_1, buf_2, buf_3)
    sems = (sem_0, sem_1, sem_2, sem_3)
    for p in range(4):
        issue(bufs[p], sems[p], p)

    def row_body(k, carry):
        r = 4 * k
        for p in range(4):
            drain(bufs[p], sems[p])
            reduce_into(bufs[p], r + p)
            issue(bufs[p], sems[p], r + p + 4)
        return carry

    lax.fori_loop(0, BPW // 4 - 1, row_body, 0)
    for p in range(4):
        drain(bufs[p], sems[p])
        reduce_into(bufs[p], BPW - 4 + p)

    pltpu.sync_copy(xt_v, xt_hbm.at[pl.ds(tbase, BPW)])


def _tc_head(xt_ref, maskT_ref, x3_ref, wT_ref, b_ref, o_ref):
    denom = jnp.clip(jnp.sum(maskT_ref[...], axis=0), 1.0, None)[:, None]
    parts = [xt_ref[...] / denom] + [x3_ref[c] for c in range(NCF)]
    x = jnp.concatenate(parts, axis=1)
    acc = lax.dot_general(wT_ref[...], x, (((0,), (1,)), ((), ())),
                          preferred_element_type=jnp.float32)
    o_ref[...] = acc + b_ref[...]


BM = 512


def kernel(encoded_text, attention_mask, categorical_vars, emb_table, cat_tables, W, b):
    emb2 = _transpose_emb(emb_table.T).reshape(NTB * TBK, D)
    cat2 = _transpose_cat(cat_tables.transpose(0, 2, 1)).reshape(NCF * CPAD * 8, CD)
    v = categorical_vars.T
    q = v // CQ
    u = v - q * CQ
    tail = q >= NQ
    row = jnp.where(tail, NQ * CS + u % CST, q * CS + (u & (CS - 1)))
    k = jnp.where(tail, u // CST, u // CS)
    cidx = (jnp.arange(NCF, dtype=jnp.int32) * (CPAD * 8))[:, None] + row * 8 + k
    t = encoded_text
    etext2 = (t >> 13) * TBK + 2 * (t & (HBK - 1)) + ((t >> 12) & 1)
    xt_sum = _sc_text(etext2, emb2)
    xc3 = _sc_cat(cidx, cat2)
    b2 = b.reshape(NCLS, 1)

    outT = pl.pallas_call(
        _tc_head,
        grid=(B // BM,),
        in_specs=[
            pl.BlockSpec((BM, D), lambda i: (i, 0)),
            pl.BlockSpec((S, BM), lambda i: (0, i)),
            pl.BlockSpec((NCF, BM, CD), lambda i: (0, i, 0)),
            pl.BlockSpec((D + NCF * CD, NCLS), lambda i: (0, 0)),
            pl.BlockSpec((NCLS, 1), lambda i: (0, 0)),
        ],
        out_specs=pl.BlockSpec((NCLS, BM), lambda i: (0, i)),
        out_shape=jax.ShapeDtypeStruct((NCLS, B), jnp.float32),
    )(xt_sum, attention_mask.T, xc3, W.T, b2)
    return outT.T


# speedup vs baseline: 5.2246x; 1.0466x over previous
import functools

import jax
import jax.numpy as jnp
from jax import lax
from jax.experimental import pallas as pl
from jax.experimental.pallas import tpu as pltpu
from jax.experimental.pallas import tpu_sc as plsc

B, S, V, D = 4096, 200, 1000000, 64
NCF, CV, CD = 26, 100000, 16
NCLS = 1000

NUM_CORES, NUM_SUBCORES = 2, 16
NW = NUM_CORES * NUM_SUBCORES
BPW = B // NW
S0, S1 = 96, 104

_sc_mesh = plsc.VectorSubcoreMesh(core_axis_name="c", subcore_axis_name="s")

TBK = 16384
HBK = TBK // 2
NTB = (V + TBK - 1) // TBK


def _tr_body(in_ref, out_ref):
    x = in_ref[...]
    xs = jnp.concatenate([x[:, :HBK], x[:, HBK:]], axis=0)
    out_ref[...] = lax.dot_general(
        xs, jnp.eye(128, dtype=jnp.float32), (((0,), (0,)), ((), ())),
        preferred_element_type=jnp.float32)


def _transpose_emb(embT):
    return pl.pallas_call(
        _tr_body,
        grid=(NTB,),
        in_specs=[pl.BlockSpec((D, TBK), lambda j: (0, j))],
        out_specs=pl.BlockSpec((HBK, 128), lambda j: (j, 0)),
        out_shape=jax.ShapeDtypeStruct((NTB * HBK, 128), jnp.float32),
    )(embT)


CQ = 4096
CS = CQ // 8
NQ = CV // CQ
CT = CV - NQ * CQ
CST = CT // 8
CPAD = NQ * CS + CST + 4


def _pack_dot(x):
    s = x.shape[1] // 8
    xs = jnp.concatenate([x[:, k * s:(k + 1) * s] for k in range(8)], axis=0)
    return lax.dot_general(xs, jnp.eye(128, dtype=jnp.float32),
                           (((0,), (0,)), ((), ())),
                           preferred_element_type=jnp.float32)


def _ctr_body(in_ref, out_ref):
    for h in range(2):
        for q in range(NQ):
            out_ref[h, pl.ds(q * CS, CS)] = _pack_dot(
                in_ref[h, :, pl.ds(q * CQ, CQ)])
        out_ref[h, pl.ds(NQ * CS, CST)] = _pack_dot(
            in_ref[h, :, pl.ds(NQ * CQ, CT)])


def _transpose_cat(catT):
    return pl.pallas_call(
        _ctr_body,
        grid=(NCF // 2,),
        in_specs=[pl.BlockSpec((2, CD, CV), lambda g: (g, 0, 0))],
        out_specs=pl.BlockSpec((2, CPAD, 128), lambda g: (g, 0, 0)),
        out_shape=jax.ShapeDtypeStruct((NCF, CPAD, 128), jnp.float32),
    )(catT)


@functools.partial(
    pl.kernel,
    out_type=jax.ShapeDtypeStruct((NCF, B, CD), jnp.float32),
    mesh=_sc_mesh,
    compiler_params=pltpu.CompilerParams(use_tc_tiling_on_sc=False),
    scratch_types=[
        pltpu.VMEM((NCF, BPW), jnp.int32),
        pltpu.VMEM((NCF, BPW, CD), jnp.float32),
        pltpu.SemaphoreType.DMA,
    ],
)
def _sc_cat(cvars_hbm, cat_hbm, xc3_hbm, cidx_v, xc_v, sem_c):
    wid = lax.axis_index("s") * NUM_CORES + lax.axis_index("c")
    tbase = wid * BPW
    pltpu.sync_copy(cvars_hbm.at[pl.ds(0, NCF), pl.ds(tbase, BPW)], cidx_v)
    cat_handles = []
    for c in range(NCF):
        cat_handles.append(pltpu.async_copy(
            cat_hbm.at[cidx_v.at[c]], xc_v.at[c], sem_c))
    for h in cat_handles:
        h.wait()
    pltpu.sync_copy(xc_v, xc3_hbm.at[pl.ds(0, NCF), pl.ds(tbase, BPW)])


@functools.partial(
    pl.kernel,
    out_type=jax.ShapeDtypeStruct((B, D), jnp.float32),
    mesh=_sc_mesh,
    compiler_params=pltpu.CompilerParams(use_tc_tiling_on_sc=False),
    scratch_types=[
        pltpu.VMEM((BPW, S), jnp.int32),
        pltpu.VMEM((S, D), jnp.float32),
        pltpu.VMEM((S, D), jnp.float32),
        pltpu.VMEM((S, D), jnp.float32),
        pltpu.VMEM((S, D), jnp.float32),
        pltpu.VMEM((BPW, D), jnp.float32),
        pltpu.SemaphoreType.DMA,
        pltpu.SemaphoreType.DMA,
        pltpu.SemaphoreType.DMA,
        pltpu.SemaphoreType.DMA,
    ],
)
def _sc_text(tidx_hbm, emb_hbm, xt_hbm,
             tidx_v, buf_0, buf_1, buf_2, buf_3, xt_v,
             sem_0, sem_1, sem_2, sem_3):
    wid = lax.axis_index("s") * NUM_CORES + lax.axis_index("c")
    tbase = wid * BPW

    pltpu.sync_copy(tidx_hbm.at[pl.ds(tbase, BPW)], tidx_v)

    def issue(buf, sem, r):
        pltpu.async_copy(emb_hbm.at[tidx_v.at[r, pl.ds(0, S0)]],
                         buf.at[pl.ds(0, S0)], sem)
        pltpu.async_copy(emb_hbm.at[tidx_v.at[r, pl.ds(S0, S1)]],
                         buf.at[pl.ds(S0, S1)], sem)

    def drain(buf, sem):
        pltpu.make_async_copy(emb_hbm.at[pl.ds(0, S0)],
                              buf.at[pl.ds(0, S0)], sem).wait()
        pltpu.make_async_copy(emb_hbm.at[pl.ds(0, S1)],
                              buf.at[pl.ds(S0, S1)], sem).wait()

    RCHUNK, NCHUNK = 25, S // 25

    def reduce_into(buf, r):
        def chunk(c, tots):
            base = c * RCHUNK
            t = list(tots)
            for g in range(RCHUNK):
                for j in range(4):
                    t[j] = t[j] + buf[base + g, pl.ds(16 * j, 16)]
            return tuple(t)

        z = jnp.zeros((16,), jnp.float32)
        tots = lax.fori_loop(0, NCHUNK, chunk, (z, z, z, z))
        for j in range(4):
            xt_v[r, pl.ds(16 * j, 16)] = tots[j]

    bufs = (buf_0, buf_1, buf_2, buf_3)
    sems = (sem_0, sem_1, sem_2, sem_3)
    for p in range(4):
        issue(bufs[p], sems[p], p)

    def row_body(k, carry):
        r = 4 * k
        for p in range(4):
            drain(bufs[p], sems[p])
            reduce_into(bufs[p], r + p)
            issue(bufs[p], sems[p], r + p + 4)
        return carry

    lax.fori_loop(0, BPW // 4 - 1, row_body, 0)
    for p in range(4):
        drain(bufs[p], sems[p])
        reduce_into(bufs[p], BPW - 4 + p)

    pltpu.sync_copy(xt_v, xt_hbm.at[pl.ds(tbase, BPW)])


def _tc_head(xt_ref, maskT_ref, x3_ref, wT_ref, b_ref, o_ref):
    denom = jnp.clip(jnp.sum(maskT_ref[...], axis=0), 1.0, None)[:, None]
    parts = [xt_ref[...] / denom] + [x3_ref[c] for c in range(NCF)]
    x = jnp.concatenate(parts, axis=1)
    acc = lax.dot_general(wT_ref[...], x, (((0,), (1,)), ((), ())),
                          preferred_element_type=jnp.float32)
    o_ref[...] = acc + b_ref[...]


BM = 512


def kernel(encoded_text, attention_mask, categorical_vars, emb_table, cat_tables, W, b):
    emb2 = _transpose_emb(emb_table.T).reshape(NTB * TBK, D)
    cat2 = _transpose_cat(cat_tables.transpose(0, 2, 1)).reshape(NCF * CPAD * 8, CD)
    v = categorical_vars.T
    q = v // CQ
    u = v - q * CQ
    tail = q >= NQ
    row = jnp.where(tail, NQ * CS + u % CST, q * CS + (u & (CS - 1)))
    k = jnp.where(tail, u // CST, u // CS)
    cidx = (jnp.arange(NCF, dtype=jnp.int32) * (CPAD * 8))[:, None] + row * 8 + k
    t = encoded_text
    etext2 = (t // TBK) * TBK + 2 * (t % HBK) + (t % TBK) // HBK
    xt_sum = _sc_text(etext2, emb2)
    xc3 = _sc_cat(cidx, cat2)
    b2 = b.reshape(NCLS, 1)

    outT = pl.pallas_call(
        _tc_head,
        grid=(B // BM,),
        in_specs=[
            pl.BlockSpec((BM, D), lambda i: (i, 0)),
            pl.BlockSpec((S, BM), lambda i: (0, i)),
            pl.BlockSpec((NCF, BM, CD), lambda i: (0, i, 0)),
            pl.BlockSpec((D + NCF * CD, NCLS), lambda i: (0, 0)),
            pl.BlockSpec((NCLS, 1), lambda i: (0, 0)),
        ],
        out_specs=pl.BlockSpec((NCLS, BM), lambda i: (0, i)),
        out_shape=jax.ShapeDtypeStruct((NCLS, B), jnp.float32),
    )(xt_sum, attention_mask.T, xc3, W.T, b2)
    return outT.T


# speedup vs baseline: 5.2787x; 1.0103x over previous
import functools

import jax
import jax.numpy as jnp
from jax import lax
from jax.experimental import pallas as pl
from jax.experimental.pallas import tpu as pltpu
from jax.experimental.pallas import tpu_sc as plsc

B, S, V, D = 4096, 200, 1000000, 64
NCF, CV, CD = 26, 100000, 16
NCLS = 1000

NUM_CORES, NUM_SUBCORES = 2, 16
NW = NUM_CORES * NUM_SUBCORES
BPW = B // NW
S0, S1 = 96, 104

_sc_mesh = plsc.VectorSubcoreMesh(core_axis_name="c", subcore_axis_name="s")

TBK = 32768
HBK = TBK // 2
NTB = (V + TBK - 1) // TBK


def _tr_body(in_ref, out_ref):
    x = in_ref[...]
    xs = jnp.concatenate([x[:, :HBK], x[:, HBK:]], axis=0)
    out_ref[...] = lax.dot_general(
        xs, jnp.eye(128, dtype=jnp.float32), (((0,), (0,)), ((), ())),
        preferred_element_type=jnp.float32)


def _transpose_emb(embT):
    return pl.pallas_call(
        _tr_body,
        grid=(NTB,),
        in_specs=[pl.BlockSpec((D, TBK), lambda j: (0, j))],
        out_specs=pl.BlockSpec((HBK, 128), lambda j: (j, 0)),
        out_shape=jax.ShapeDtypeStruct((NTB * HBK, 128), jnp.float32),
    )(embT)


CQ = 8192
CS = CQ // 8
NQ = CV // CQ
CT = CV - NQ * CQ
CST = CT // 8
CPAD = NQ * CS + CST + 4


def _pack_dot(x):
    s = x.shape[1] // 8
    xs = jnp.concatenate([x[:, k * s:(k + 1) * s] for k in range(8)], axis=0)
    return lax.dot_general(xs, jnp.eye(128, dtype=jnp.float32),
                           (((0,), (0,)), ((), ())),
                           preferred_element_type=jnp.float32)


def _ctr_body(in_ref, out_ref):
    for h in range(2):
        for q in range(NQ):
            out_ref[h, pl.ds(q * CS, CS)] = _pack_dot(
                in_ref[h, :, pl.ds(q * CQ, CQ)])
        out_ref[h, pl.ds(NQ * CS, CST)] = _pack_dot(
            in_ref[h, :, pl.ds(NQ * CQ, CT)])


def _transpose_cat(catT):
    return pl.pallas_call(
        _ctr_body,
        grid=(NCF // 2,),
        in_specs=[pl.BlockSpec((2, CD, CV), lambda g: (g, 0, 0))],
        out_specs=pl.BlockSpec((2, CPAD, 128), lambda g: (g, 0, 0)),
        out_shape=jax.ShapeDtypeStruct((NCF, CPAD, 128), jnp.float32),
    )(catT)


@functools.partial(
    pl.kernel,
    out_type=jax.ShapeDtypeStruct((NCF, B, CD), jnp.float32),
    mesh=_sc_mesh,
    compiler_params=pltpu.CompilerParams(use_tc_tiling_on_sc=False),
    scratch_types=[
        pltpu.VMEM((NCF, BPW), jnp.int32),
        pltpu.VMEM((NCF, BPW, CD), jnp.float32),
        pltpu.SemaphoreType.DMA,
    ],
)
def _sc_cat(cvars_hbm, cat_hbm, xc3_hbm, cidx_v, xc_v, sem_c):
    wid = lax.axis_index("s") * NUM_CORES + lax.axis_index("c")
    tbase = wid * BPW
    pltpu.sync_copy(cvars_hbm.at[pl.ds(0, NCF), pl.ds(tbase, BPW)], cidx_v)
    cat_handles = []
    for c in range(NCF):
        cat_handles.append(pltpu.async_copy(
            cat_hbm.at[cidx_v.at[c]], xc_v.at[c], sem_c))
    for h in cat_handles:
        h.wait()
    pltpu.sync_copy(xc_v, xc3_hbm.at[pl.ds(0, NCF), pl.ds(tbase, BPW)])


@functools.partial(
    pl.kernel,
    out_type=jax.ShapeDtypeStruct((B, D), jnp.float32),
    mesh=_sc_mesh,
    compiler_params=pltpu.CompilerParams(use_tc_tiling_on_sc=False),
    scratch_types=[
        pltpu.VMEM((BPW, S), jnp.int32),
        pltpu.VMEM((S, D), jnp.float32),
        pltpu.VMEM((S, D), jnp.float32),
        pltpu.VMEM((S, D), jnp.float32),
        pltpu.VMEM((S, D), jnp.float32),
        pltpu.VMEM((BPW, D), jnp.float32),
        pltpu.SemaphoreType.DMA,
        pltpu.SemaphoreType.DMA,
        pltpu.SemaphoreType.DMA,
        pltpu.SemaphoreType.DMA,
    ],
)
def _sc_text(tidx_hbm, emb_hbm, xt_hbm,
             tidx_v, buf_0, buf_1, buf_2, buf_3, xt_v,
             sem_0, sem_1, sem_2, sem_3):
    wid = lax.axis_index("s") * NUM_CORES + lax.axis_index("c")
    tbase = wid * BPW

    pltpu.sync_copy(tidx_hbm.at[pl.ds(tbase, BPW)], tidx_v)

    def issue(buf, sem, r):
        pltpu.async_copy(emb_hbm.at[tidx_v.at[r, pl.ds(0, S0)]],
                         buf.at[pl.ds(0, S0)], sem)
        pltpu.async_copy(emb_hbm.at[tidx_v.at[r, pl.ds(S0, S1)]],
                         buf.at[pl.ds(S0, S1)], sem)

    def drain(buf, sem):
        pltpu.make_async_copy(emb_hbm.at[pl.ds(0, S0)],
                              buf.at[pl.ds(0, S0)], sem).wait()
        pltpu.make_async_copy(emb_hbm.at[pl.ds(0, S1)],
                              buf.at[pl.ds(S0, S1)], sem).wait()

    RCHUNK, NCHUNK = 25, S // 25

    def reduce_into(buf, r):
        def chunk(c, tots):
            base = c * RCHUNK
            t = list(tots)
            for g in range(RCHUNK):
                for j in range(4):
                    t[j] = t[j] + buf[base + g, pl.ds(16 * j, 16)]
            return tuple(t)

        z = jnp.zeros((16,), jnp.float32)
        tots = lax.fori_loop(0, NCHUNK, chunk, (z, z, z, z))
        for j in range(4):
            xt_v[r, pl.ds(16 * j, 16)] = tots[j]

    bufs = (buf_0, buf_1, buf_2, buf_3)
    sems = (sem_0, sem_1, sem_2, sem_3)
    for p in range(4):
        issue(bufs[p], sems[p], p)

    def row_body(k, carry):
        r = 4 * k
        for p in range(4):
            drain(bufs[p], sems[p])
            reduce_into(bufs[p], r + p)
            issue(bufs[p], sems[p], r + p + 4)
        return carry

    lax.fori_loop(0, BPW // 4 - 1, row_body, 0)
    for p in range(4):
        drain(bufs[p], sems[p])
        reduce_into(bufs[p], BPW - 4 + p)

    pltpu.sync_copy(xt_v, xt_hbm.at[pl.ds(tbase, BPW)])


def _tc_head(xt_ref, maskT_ref, x3_ref, wT_ref, b_ref, o_ref):
    denom = jnp.clip(jnp.sum(maskT_ref[...], axis=0), 1.0, None)[:, None]
    parts = [xt_ref[...] / denom] + [x3_ref[c] for c in range(NCF)]
    x = jnp.concatenate(parts, axis=1)
    acc = lax.dot_general(wT_ref[...], x, (((0,), (1,)), ((), ())),
                          preferred_element_type=jnp.float32)
    o_ref[...] = acc + b_ref[...]


BM = 512


def kernel(encoded_text, attention_mask, categorical_vars, emb_table, cat_tables, W, b):
    emb2 = _transpose_emb(emb_table.T).reshape(NTB * TBK, D)
    cat2 = _transpose_cat(cat_tables.transpose(0, 2, 1)).reshape(NCF * CPAD * 8, CD)
    v = categorical_vars.T
    q = v // CQ
    u = v - q * CQ
    tail = q >= NQ
    row = jnp.where(tail, NQ * CS + u % CST, q * CS + (u & (CS - 1)))
    k = jnp.where(tail, u // CST, u // CS)
    cidx = (jnp.arange(NCF, dtype=jnp.int32) * (CPAD * 8))[:, None] + row * 8 + k
    t = encoded_text
    etext2 = (t // TBK) * TBK + 2 * (t % HBK) + (t % TBK) // HBK
    xt_sum = _sc_text(etext2, emb2)
    xc3 = _sc_cat(cidx, cat2)
    b2 = b.reshape(NCLS, 1)

    outT = pl.pallas_call(
        _tc_head,
        grid=(B // BM,),
        in_specs=[
            pl.BlockSpec((BM, D), lambda i: (i, 0)),
            pl.BlockSpec((S, BM), lambda i: (0, i)),
            pl.BlockSpec((NCF, BM, CD), lambda i: (0, i, 0)),
            pl.BlockSpec((D + NCF * CD, NCLS), lambda i: (0, 0)),
            pl.BlockSpec((NCLS, 1), lambda i: (0, 0)),
        ],
        out_specs=pl.BlockSpec((NCLS, BM), lambda i: (0, i)),
        out_shape=jax.ShapeDtypeStruct((NCLS, B), jnp.float32),
    )(xt_sum, attention_mask.T, xc3, W.T, b2)
    return outT.T
